# baseline (speedup 1.0000x reference)
import functools

import jax
import jax.numpy as jnp
from jax import lax
from jax.experimental import pallas as pl
from jax.experimental.pallas import tpu as pltpu
from jax.experimental.pallas import tpu_sc as plsc

D_MODEL = 128
SEQ = 64
PERIOD = SEQ * D_MODEL
_INFO = plsc.get_sparse_core_info()
_NC = _INFO.num_cores
_NW = _INFO.num_cores * _INFO.num_subcores

TOTAL = 4096 * SEQ * D_MODEL
TILE_ELEMS = TOTAL // _NW
CHUNK = 4 * PERIOD
NCHUNKS = TILE_ELEMS // CHUNK


def _sc_add(x_flat, inds, pe64):
    mesh = plsc.VectorSubcoreMesh(core_axis_name="c", subcore_axis_name="s")

    @functools.partial(
        pl.kernel,
        mesh=mesh,
        out_type=jax.ShapeDtypeStruct((TOTAL,), jnp.float32),
        scratch_types=[
            pltpu.VMEM((SEQ,), jnp.int32),
            pltpu.VMEM((SEQ, D_MODEL), jnp.float32),
            pltpu.VMEM((CHUNK,), jnp.float32),
            pltpu.VMEM((CHUNK,), jnp.float32),
            pltpu.SemaphoreType.DMA,
            pltpu.SemaphoreType.DMA,
            pltpu.SemaphoreType.DMA,
            pltpu.SemaphoreType.DMA,
            pltpu.SemaphoreType.DMA,
        ],
    )
    def body(x_hbm, idx_hbm, pe_hbm, out_hbm, idx_v, fp, buf0, buf1,
             rsem0, rsem1, wsem0, wsem1, gsem):
        wid = lax.axis_index("s") * _NC + lax.axis_index("c")
        base_t = wid * TILE_ELEMS

        pltpu.sync_copy(idx_hbm, idx_v)
        pltpu.async_copy(pe_hbm.at[idx_v], fp, gsem).wait()

        bufs = (buf0, buf1)
        rsems = (rsem0, rsem1)
        wsems = (wsem0, wsem1)

        def start_read(i):
            return pltpu.async_copy(
                x_hbm.at[pl.ds(base_t + i * CHUNK, CHUNK)],
                bufs[i % 2], rsems[i % 2])

        def add_chunk(buf):
            def inner(l2, _):
                l = lax.rem(l2, SEQ)
                for k in range(8):
                    off = l2 * D_MODEL + k * 16
                    buf[pl.ds(off, 16)] = (
                        buf[pl.ds(off, 16)] + fp[l, pl.ds(k * 16, 16)])
                return 0
            lax.fori_loop(0, CHUNK // D_MODEL, inner, 0)

        rhandles = [start_read(0), None]
        whandles = [None, None]
        for i in range(NCHUNKS):
            b = i % 2
            if i + 1 < NCHUNKS:
                nb = (i + 1) % 2
                if whandles[nb] is not None:
                    whandles[nb].wait()
                rhandles[nb] = start_read(i + 1)
            rhandles[b].wait()
            add_chunk(bufs[b])
            whandles[b] = pltpu.async_copy(
                bufs[b],
                out_hbm.at[pl.ds(base_t + i * CHUNK, CHUNK)],
                wsems[b])
        whandles[0].wait()
        whandles[1].wait()

    return body(x_flat, inds, pe64)


def kernel(x, x_node_inds, pe):
    inds = x_node_inds.astype(jnp.int32)
    pe64 = pe[:SEQ]
    out_flat = _sc_add(x.reshape(-1), inds, pe64)
    return out_flat.reshape(x.shape)


# speedup vs baseline: 4.3326x; 4.3326x over previous
import jax
import jax.numpy as jnp
from jax.experimental import pallas as pl
from jax.experimental.pallas import tpu as pltpu

D_MODEL = 128
SEQ = 64
BATCH_BLOCK = 256


def _body(inds_ref, pe_ref, x_ref, o_ref, fp_ref):
    @pl.when(pl.program_id(0) == 0)
    def _():
        iota = jax.lax.broadcasted_iota(jnp.int32, (SEQ, SEQ), 1)
        onehot = (inds_ref[...] == iota).astype(jnp.float32)
        fp_ref[...] = jnp.dot(
            onehot, pe_ref[...], preferred_element_type=jnp.float32)

    o_ref[...] = x_ref[...] + fp_ref[...][None, :, :]


def kernel(x, x_node_inds, pe):
    nb = x.shape[0] // BATCH_BLOCK
    inds2d = x_node_inds.astype(jnp.int32).reshape(SEQ, 1)
    pe64 = pe[:SEQ]

    return pl.pallas_call(
        _body,
        grid=(nb,),
        in_specs=[
            pl.BlockSpec((SEQ, 1), lambda i: (0, 0)),
            pl.BlockSpec((SEQ, D_MODEL), lambda i: (0, 0)),
            pl.BlockSpec((BATCH_BLOCK, SEQ, D_MODEL), lambda i: (i, 0, 0)),
        ],
        out_specs=pl.BlockSpec((BATCH_BLOCK, SEQ, D_MODEL), lambda i: (i, 0, 0)),
        out_shape=jax.ShapeDtypeStruct(x.shape, x.dtype),
        scratch_shapes=[pltpu.VMEM((SEQ, D_MODEL), jnp.float32)],
        compiler_params=pltpu.CompilerParams(
            dimension_semantics=("arbitrary",),
        ),
    )(inds2d, pe64, x)


# speedup vs baseline: 4.3690x; 1.0084x over previous
import jax
import jax.numpy as jnp
from jax.experimental import pallas as pl
from jax.experimental.pallas import tpu as pltpu

D_MODEL = 128
SEQ = 64
BATCH_BLOCK = 256


def _body(inds_ref, x_ref, pe_ref, o_ref, fp_ref):
    @pl.when(pl.program_id(0) == 0)
    def _():
        def gather_row(j, _):
            idx = inds_ref[j]
            fp_ref[pl.ds(j, 1), :] = pe_ref[pl.ds(idx, 1), :]
            return 0

        jax.lax.fori_loop(0, SEQ, gather_row, 0)

    o_ref[...] = x_ref[...] + fp_ref[...][None, :, :]


def kernel(x, x_node_inds, pe):
    nb = x.shape[0] // BATCH_BLOCK
    inds = x_node_inds.astype(jnp.int32)
    pe64 = pe[:SEQ]

    grid_spec = pltpu.PrefetchScalarGridSpec(
        num_scalar_prefetch=1,
        grid=(nb,),
        in_specs=[
            pl.BlockSpec((BATCH_BLOCK, SEQ, D_MODEL), lambda i, inds_ref: (i, 0, 0)),
            pl.BlockSpec((SEQ, D_MODEL), lambda i, inds_ref: (0, 0)),
        ],
        out_specs=pl.BlockSpec((BATCH_BLOCK, SEQ, D_MODEL), lambda i, inds_ref: (i, 0, 0)),
        scratch_shapes=[pltpu.VMEM((SEQ, D_MODEL), jnp.float32)],
    )

    return pl.pallas_call(
        _body,
        grid_spec=grid_spec,
        out_shape=jax.ShapeDtypeStruct(x.shape, x.dtype),
        compiler_params=pltpu.CompilerParams(
            dimension_semantics=("arbitrary",),
        ),
    )(inds, x, pe64)
